# baseline (speedup 1.0000x reference)
import jax
import jax.numpy as jnp
from jax.experimental import pallas as pl
from jax.experimental.pallas import tpu as pltpu

B = 1024
P = 32
N = 20
NP = N * P
D = 6144
N_PAD = 32
K = NP + N_PAD
DBLK = 1024
NBLK = D // DBLK
NSLOT = 3
CSPLIT = 160
OSPLIT = 256


def _decode_kernel(lab_ref, lat_ref, comp_hbm, means_hbm, out_hbm,
                   e_ref, w_buf, out_buf, csem, msem, out_sem):
    def in_copies(i, slot):
        return [
            pltpu.make_async_copy(
                comp_hbm.at[pl.ds(p * CSPLIT, CSPLIT), pl.ds(i * DBLK, DBLK)],
                w_buf.at[slot, pl.ds(p * CSPLIT, CSPLIT)],
                csem.at[slot, p],
            )
            for p in range(NP // CSPLIT)
        ] + [
            pltpu.make_async_copy(
                means_hbm.at[:, pl.ds(i * DBLK, DBLK)],
                w_buf.at[slot, pl.ds(NP, N_PAD)],
                msem.at[slot],
            )
        ]

    def out_copies(i, slot):
        return [
            pltpu.make_async_copy(
                out_buf.at[slot, pl.ds(p * OSPLIT, OSPLIT)],
                out_hbm.at[pl.ds(p * OSPLIT, OSPLIT), pl.ds(i * DBLK, DBLK)],
                out_sem.at[slot, p],
            )
            for p in range(B // OSPLIT)
        ]

    for c in in_copies(0, 0):
        c.start()

    lab = lab_ref[:, :1]
    j = jax.lax.broadcasted_iota(jnp.int32, (B, K), 1)
    cls = jnp.where(j < NP, j // P, j - NP)
    lat_t = jnp.concatenate([lat_ref[...]] * (K // P), axis=1)
    val = jnp.where(j < NP, lat_t, 1.0)
    e_ref[...] = jnp.where(cls == lab, val, 0.0).astype(jnp.bfloat16)

    for c in in_copies(1, 1):
        c.start()

    for i in range(NBLK):
        slot = i % NSLOT
        if i + 2 < NBLK:
            for c in in_copies(i + 2, (i + 2) % NSLOT):
                c.start()
        for c in in_copies(i, slot):
            c.wait()
        if i >= NSLOT:
            for c in out_copies(i - NSLOT, slot):
                c.wait()
        out_buf[slot] = jnp.dot(
            e_ref[...],
            w_buf[slot].astype(jnp.bfloat16),
            preferred_element_type=jnp.float32,
        )
        for c in out_copies(i, slot):
            c.start()

    for i in range(max(0, NBLK - NSLOT), NBLK):
        for c in out_copies(i, i % NSLOT):
            c.wait()


def kernel(lattent_codes, object_labels, means, components):
    comp2d = components.reshape(NP, D)
    labels_2d = object_labels.astype(jnp.int32)[:, None]
    means_pad = jnp.pad(means, ((0, N_PAD - N), (0, 0)))

    return pl.pallas_call(
        _decode_kernel,
        grid=(1,),
        in_specs=[
            pl.BlockSpec((B, 1), lambda i: (0, 0)),
            pl.BlockSpec((B, P), lambda i: (0, 0)),
            pl.BlockSpec(memory_space=pl.ANY),
            pl.BlockSpec(memory_space=pl.ANY),
        ],
        out_specs=pl.BlockSpec(memory_space=pl.ANY),
        out_shape=jax.ShapeDtypeStruct((B, D), jnp.float32),
        scratch_shapes=[
            pltpu.VMEM((B, K), jnp.bfloat16),
            pltpu.VMEM((NSLOT, K, DBLK), jnp.float32),
            pltpu.VMEM((NSLOT, B, DBLK), jnp.float32),
            pltpu.SemaphoreType.DMA((NSLOT, NP // CSPLIT)),
            pltpu.SemaphoreType.DMA((NSLOT,)),
            pltpu.SemaphoreType.DMA((NSLOT, B // OSPLIT)),
        ],
    )(labels_2d, lattent_codes, comp2d, means_pad)


# speedup vs baseline: 1.0029x; 1.0029x over previous
import jax
import jax.numpy as jnp
from jax.experimental import pallas as pl
from jax.experimental.pallas import tpu as pltpu

B = 1024
P = 32
N = 20
NP = N * P
D = 6144
N_PAD = 32
K = NP + N_PAD
DBLK = 1024
NBLK = D // DBLK
NSLOT = 3
CSPLIT = 320
OSPLIT = 512


def _decode_kernel(lab_ref, lat_ref, comp_hbm, means_hbm, out_hbm,
                   e_ref, w_buf, out_buf, csem, msem, out_sem):
    def in_copies(i, slot):
        return [
            pltpu.make_async_copy(
                comp_hbm.at[pl.ds(p * CSPLIT, CSPLIT), pl.ds(i * DBLK, DBLK)],
                w_buf.at[slot, pl.ds(p * CSPLIT, CSPLIT)],
                csem.at[slot, p],
            )
            for p in range(NP // CSPLIT)
        ] + [
            pltpu.make_async_copy(
                means_hbm.at[:, pl.ds(i * DBLK, DBLK)],
                w_buf.at[slot, pl.ds(NP, N_PAD)],
                msem.at[slot],
            )
        ]

    def out_copies(i, slot):
        return [
            pltpu.make_async_copy(
                out_buf.at[slot, pl.ds(p * OSPLIT, OSPLIT)],
                out_hbm.at[pl.ds(p * OSPLIT, OSPLIT), pl.ds(i * DBLK, DBLK)],
                out_sem.at[slot, p],
            )
            for p in range(B // OSPLIT)
        ]

    for c in in_copies(0, 0):
        c.start()

    lab = lab_ref[:, :1]
    j = jax.lax.broadcasted_iota(jnp.int32, (B, K), 1)
    cls = jnp.where(j < NP, j // P, j - NP)
    lat_t = jnp.concatenate([lat_ref[...]] * (K // P), axis=1)
    val = jnp.where(j < NP, lat_t, 1.0)
    e_ref[...] = jnp.where(cls == lab, val, 0.0)

    for c in in_copies(1, 1):
        c.start()

    for i in range(NBLK):
        slot = i % NSLOT
        if i + 2 < NBLK:
            for c in in_copies(i + 2, (i + 2) % NSLOT):
                c.start()
        for c in in_copies(i, slot):
            c.wait()
        if i >= NSLOT:
            for c in out_copies(i - NSLOT, slot):
                c.wait()
        out_buf[slot] = jnp.dot(
            e_ref[...], w_buf[slot], preferred_element_type=jnp.float32
        )
        for c in out_copies(i, slot):
            c.start()

    for i in range(max(0, NBLK - NSLOT), NBLK):
        for c in out_copies(i, i % NSLOT):
            c.wait()


def kernel(lattent_codes, object_labels, means, components):
    comp2d = components.reshape(NP, D)
    labels_2d = object_labels.astype(jnp.int32)[:, None]
    means_pad = jnp.pad(means, ((0, N_PAD - N), (0, 0)))

    return pl.pallas_call(
        _decode_kernel,
        grid=(1,),
        in_specs=[
            pl.BlockSpec((B, 1), lambda i: (0, 0)),
            pl.BlockSpec((B, P), lambda i: (0, 0)),
            pl.BlockSpec(memory_space=pl.ANY),
            pl.BlockSpec(memory_space=pl.ANY),
        ],
        out_specs=pl.BlockSpec(memory_space=pl.ANY),
        out_shape=jax.ShapeDtypeStruct((B, D), jnp.float32),
        scratch_shapes=[
            pltpu.VMEM((B, K), jnp.float32),
            pltpu.VMEM((NSLOT, K, DBLK), jnp.float32),
            pltpu.VMEM((NSLOT, B, DBLK), jnp.float32),
            pltpu.SemaphoreType.DMA((NSLOT, NP // CSPLIT)),
            pltpu.SemaphoreType.DMA((NSLOT,)),
            pltpu.SemaphoreType.DMA((NSLOT, B // OSPLIT)),
        ],
    )(labels_2d, lattent_codes, comp2d, means_pad)


# speedup vs baseline: 1.0123x; 1.0093x over previous
import jax
import jax.numpy as jnp
from jax.experimental import pallas as pl
from jax.experimental.pallas import tpu as pltpu

B = 1024
P = 32
N = 20
NP = N * P
D = 6144
N_PAD = 32
K = NP + N_PAD
DBLK = 1024
NBLK = D // DBLK
NSLOT = 4
CSPLIT = 320
OSPLIT = 512


def _decode_kernel(lab_ref, lat_ref, comp_hbm, means_hbm, out_hbm,
                   e_ref, w_buf, out_buf, csem, msem, out_sem):
    def in_copies(i, slot):
        return [
            pltpu.make_async_copy(
                comp_hbm.at[pl.ds(p * CSPLIT, CSPLIT), pl.ds(i * DBLK, DBLK)],
                w_buf.at[slot, pl.ds(p * CSPLIT, CSPLIT)],
                csem.at[slot, p],
            )
            for p in range(NP // CSPLIT)
        ] + [
            pltpu.make_async_copy(
                means_hbm.at[:, pl.ds(i * DBLK, DBLK)],
                w_buf.at[slot, pl.ds(NP, N_PAD)],
                msem.at[slot],
            )
        ]

    def out_copies(i, slot):
        return [
            pltpu.make_async_copy(
                out_buf.at[slot, pl.ds(p * OSPLIT, OSPLIT)],
                out_hbm.at[pl.ds(p * OSPLIT, OSPLIT), pl.ds(i * DBLK, DBLK)],
                out_sem.at[slot, p],
            )
            for p in range(B // OSPLIT)
        ]

    for c in in_copies(0, 0):
        c.start()

    lab = lab_ref[:, :1]
    j = jax.lax.broadcasted_iota(jnp.int32, (B, K), 1)
    cls = jnp.where(j < NP, j // P, j - NP)
    lat_t = jnp.concatenate([lat_ref[...]] * (K // P), axis=1)
    val = jnp.where(j < NP, lat_t, 1.0)
    e_ref[...] = jnp.where(cls == lab, val, 0.0)

    for c in in_copies(1, 1):
        c.start()
    for c in in_copies(2, 2):
        c.start()

    for i in range(NBLK):
        slot = i % NSLOT
        if i + 3 < NBLK:
            for c in in_copies(i + 3, (i + 3) % NSLOT):
                c.start()
        for c in in_copies(i, slot):
            c.wait()
        if i >= NSLOT:
            for c in out_copies(i - NSLOT, slot):
                c.wait()
        out_buf[slot] = jnp.dot(
            e_ref[...], w_buf[slot], preferred_element_type=jnp.float32
        )
        for c in out_copies(i, slot):
            c.start()

    for i in range(max(0, NBLK - NSLOT), NBLK):
        for c in out_copies(i, i % NSLOT):
            c.wait()


def kernel(lattent_codes, object_labels, means, components):
    comp2d = components.reshape(NP, D)
    labels_2d = object_labels.astype(jnp.int32)[:, None]
    means_pad = jnp.pad(means, ((0, N_PAD - N), (0, 0)))

    return pl.pallas_call(
        _decode_kernel,
        grid=(1,),
        in_specs=[
            pl.BlockSpec((B, 1), lambda i: (0, 0)),
            pl.BlockSpec((B, P), lambda i: (0, 0)),
            pl.BlockSpec(memory_space=pl.ANY),
            pl.BlockSpec(memory_space=pl.ANY),
        ],
        out_specs=pl.BlockSpec(memory_space=pl.ANY),
        out_shape=jax.ShapeDtypeStruct((B, D), jnp.float32),
        scratch_shapes=[
            pltpu.VMEM((B, K), jnp.float32),
            pltpu.VMEM((NSLOT, K, DBLK), jnp.float32),
            pltpu.VMEM((NSLOT, B, DBLK), jnp.float32),
            pltpu.SemaphoreType.DMA((NSLOT, NP // CSPLIT)),
            pltpu.SemaphoreType.DMA((NSLOT,)),
            pltpu.SemaphoreType.DMA((NSLOT, B // OSPLIT)),
        ],
    )(labels_2d, lattent_codes, comp2d, means_pad)
